# baseline (speedup 1.0000x reference)
import jax
import jax.numpy as jnp
from jax.experimental import pallas as pl
from jax.experimental.pallas import tpu as pltpu


def kernel(v, n_feats, aud, n_auds, o, n_answ, temp):
    B, VD = v.shape
    N = n_feats.shape[0]
    LD = aud.shape[1]
    OD = n_answ.shape[1]
    NB = 4
    BK = N // NB
    KSEL = float(min(1000, N))

    def body(v_ref, nf_ref, aud_ref, na_ref, o_ref, nans_ref, temp_ref,
             out_ref, qn1, qn2, e1, e2, z1, z2, att, acc):
        p = pl.program_id(0)
        i = pl.program_id(1)

        @pl.when(p == 0)
        def _phase0():
            @pl.when(i == 0)
            def _init():
                vv = v_ref[...]
                nv = jnp.maximum(
                    jnp.sqrt(jnp.sum(vv * vv, axis=1, keepdims=True)), 1e-12)
                qn1[...] = vv / nv
                av = aud_ref[...]
                na_ = jnp.maximum(
                    jnp.sqrt(jnp.sum(av * av, axis=1, keepdims=True)), 1e-12)
                qn2[...] = av / na_
                z1[...] = jnp.zeros_like(z1)
                z2[...] = jnp.zeros_like(z2)

            k1 = nf_ref[...]
            ss1 = jax.lax.dot_general(
                k1 * k1, jnp.ones((VD, 8), jnp.float32),
                (((1,), (0,)), ((), ())),
                preferred_element_type=jnp.float32)
            r1 = 1.0 / jnp.maximum(jnp.sqrt(ss1), 1e-12)
            r1row = jnp.transpose(r1, (1, 0))[:1, :]
            s1 = jax.lax.dot_general(qn1[...], k1, (((1,), (1,)), ((), ())),
                                     preferred_element_type=jnp.float32)
            eb1 = jnp.exp(jnp.clip(s1 * r1row, 0.0, 1.0))
            e1[:, pl.ds(i * BK, BK)] = eb1
            z1[...] += jnp.sum(eb1, axis=1, keepdims=True)

            k2 = na_ref[...]
            ss2 = jax.lax.dot_general(
                k2 * k2, jnp.ones((LD, 8), jnp.float32),
                (((1,), (0,)), ((), ())),
                preferred_element_type=jnp.float32)
            r2 = 1.0 / jnp.maximum(jnp.sqrt(ss2), 1e-12)
            r2row = jnp.transpose(r2, (1, 0))[:1, :]
            s2 = jax.lax.dot_general(qn2[...], k2, (((1,), (1,)), ((), ())),
                                     preferred_element_type=jnp.float32)
            eb2 = jnp.exp(jnp.clip(s2 * r2row, 0.0, 1.0))
            e2[:, pl.ds(i * BK, BK)] = eb2
            z2[...] += jnp.sum(eb2, axis=1, keepdims=True)

        @pl.when(p == 1)
        def _phase1():
            @pl.when(i == 0)
            def _select():
                a = jax.nn.sigmoid(temp_ref[0, 0])
                c1 = a / z1[...]
                c2 = (1.0 - a) / z2[...]
                attv = e1[...] * c1 + e2[...] * c2
                bits = jax.lax.bitcast_convert_type(attv, jnp.int32)
                mx = jnp.max(attv, axis=1, keepdims=True)
                mn = jnp.min(attv, axis=1, keepdims=True)
                hi0 = jax.lax.bitcast_convert_type(mx, jnp.int32) + 1
                lo0 = jax.lax.bitcast_convert_type(mn, jnp.int32)
                H = N // 2
                base = lo0

                def pack_pairs(x):
                    return x[:, :H] | jax.lax.shift_left(x[:, H:], 16)

                def swar_count(qp, mid):
                    adder = (1 << 15) - mid
                    adder = adder | jax.lax.shift_left(adder, 16)
                    t = qp + adder
                    m = jax.lax.shift_right_logical(t, 15) & 0x00010001
                    w = H
                    while w > 128:
                        w //= 2
                        m = m[:, :w] + m[:, w:]
                    cp = jnp.sum(m, axis=1, keepdims=True)
                    return (cp & 0xFFFF) + jax.lax.shift_right_logical(
                        cp, 16)

                def cond(lohi):
                    lo, hi = lohi
                    return jnp.max(hi - lo) > 1

                qa = pack_pairs(jax.lax.shift_right_logical(bits - base, 10))
                qhi0 = jax.lax.shift_right_logical(hi0 - 1 - base, 10) + 1
                qlo0 = jnp.zeros_like(qhi0)

                def ternary_step(qp, lo, hi, off):
                    d = hi - lo
                    t1 = lo + jnp.maximum(d // 3, 1)
                    t2 = jnp.clip(lo + (2 * d) // 3, t1, hi - 1)
                    ge1 = swar_count(qp, t1 - off) >= int(KSEL)
                    ge2 = swar_count(qp, t2 - off) >= int(KSEL)
                    lo = jnp.where(ge2, t2, jnp.where(ge1, t1, lo))
                    hi = jnp.where(ge2, hi, jnp.where(ge1, t2, t1))
                    return lo, hi

                qlo, _ = jax.lax.while_loop(
                    cond, lambda lh: ternary_step(qa, lh[0], lh[1], 0),
                    (qlo0, qhi0))
                lob = base + jax.lax.shift_left(qlo, 10)
                hib = jnp.minimum(
                    base + jax.lax.shift_left(qlo + 1, 10), hi0)

                qb = pack_pairs(jnp.clip(bits - lob, 0, 4095))

                lo, _ = jax.lax.while_loop(
                    cond, lambda lh: ternary_step(qb, lh[0], lh[1], lob),
                    (lob, hib))
                keep = (bits >= lo).astype(jnp.float32)
                colmask = jnp.max(keep, axis=0, keepdims=True)
                att[...] = jax.lax.bitcast_convert_type(
                    bits, jnp.float32) * colmask
                acc[...] = jnp.zeros_like(acc)

            ablk = att[:, pl.ds(i * BK, BK)]
            acc[...] += jax.lax.dot_general(
                ablk, nans_ref[...], (((1,), (0,)), ((), ())),
                preferred_element_type=jnp.float32)

            @pl.when(i == NB - 1)
            def _final():
                oa = acc[...]
                s0 = jnp.sum(oa * o_ref[:, 0, :], axis=1, keepdims=True)
                s1_ = jnp.sum(oa * o_ref[:, 1, :], axis=1, keepdims=True)
                s2_ = jnp.sum(oa * o_ref[:, 2, :], axis=1, keepdims=True)
                out_ref[...] = jnp.concatenate([s0, s1_, s2_], axis=1)

    out = pl.pallas_call(
        body,
        grid=(2, NB),
        in_specs=[
            pl.BlockSpec((B, VD), lambda p, i: (0, 0)),
            pl.BlockSpec((BK, VD), lambda p, i: ((1 - p) * i, 0)),
            pl.BlockSpec((B, LD), lambda p, i: (0, 0)),
            pl.BlockSpec((BK, LD), lambda p, i: ((1 - p) * i, 0)),
            pl.BlockSpec((B, 3, OD), lambda p, i: (0, 0, 0)),
            pl.BlockSpec((BK, OD), lambda p, i: (p * i, 0)),
            pl.BlockSpec((1, 1), lambda p, i: (0, 0)),
        ],
        out_specs=pl.BlockSpec((B, 3), lambda p, i: (0, 0)),
        out_shape=jax.ShapeDtypeStruct((B, 3), jnp.float32),
        scratch_shapes=[
            pltpu.VMEM((B, VD), jnp.float32),
            pltpu.VMEM((B, LD), jnp.float32),
            pltpu.VMEM((B, N), jnp.float32),
            pltpu.VMEM((B, N), jnp.float32),
            pltpu.VMEM((B, 1), jnp.float32),
            pltpu.VMEM((B, 1), jnp.float32),
            pltpu.VMEM((B, N), jnp.float32),
            pltpu.VMEM((B, OD), jnp.float32),
        ],
    )(v, n_feats, aud, n_auds, o, n_answ, temp.reshape(1, 1))
    return out


# speedup vs baseline: 1.0541x; 1.0541x over previous
import jax
import jax.numpy as jnp
from jax.experimental import pallas as pl
from jax.experimental.pallas import tpu as pltpu


def kernel(v, n_feats, aud, n_auds, o, n_answ, temp):
    B, VD = v.shape
    N = n_feats.shape[0]
    LD = aud.shape[1]
    OD = n_answ.shape[1]
    NB = 4
    BK = N // NB
    KSEL = float(min(1000, N))

    def body(v_ref, nf_ref, aud_ref, na_ref, o_ref, nans_ref, temp_ref,
             out_ref, qn1, qn2, e1, e2, z1, z2, att, acc):
        p = pl.program_id(0)
        i = pl.program_id(1)

        @pl.when(p == 0)
        def _phase0():
            @pl.when(i == 0)
            def _init():
                vv = v_ref[...]
                nv = jnp.maximum(
                    jnp.sqrt(jnp.sum(vv * vv, axis=1, keepdims=True)), 1e-12)
                qn1[...] = vv / nv
                av = aud_ref[...]
                na_ = jnp.maximum(
                    jnp.sqrt(jnp.sum(av * av, axis=1, keepdims=True)), 1e-12)
                qn2[...] = av / na_
                z1[...] = jnp.zeros_like(z1)
                z2[...] = jnp.zeros_like(z2)

            k1 = nf_ref[...]
            ss1 = jax.lax.dot_general(
                k1 * k1, jnp.ones((VD, 8), jnp.float32),
                (((1,), (0,)), ((), ())),
                preferred_element_type=jnp.float32)
            r1 = 1.0 / jnp.maximum(jnp.sqrt(ss1), 1e-12)
            r1row = jnp.transpose(r1, (1, 0))[:1, :]
            s1 = jax.lax.dot_general(qn1[...], k1, (((1,), (1,)), ((), ())),
                                     preferred_element_type=jnp.float32)
            eb1 = jnp.exp(jnp.clip(s1 * r1row, 0.0, 1.0))
            e1[:, pl.ds(i * BK, BK)] = eb1
            z1[...] += jnp.sum(eb1, axis=1, keepdims=True)

            k2 = na_ref[...]
            ss2 = jax.lax.dot_general(
                k2 * k2, jnp.ones((LD, 8), jnp.float32),
                (((1,), (0,)), ((), ())),
                preferred_element_type=jnp.float32)
            r2 = 1.0 / jnp.maximum(jnp.sqrt(ss2), 1e-12)
            r2row = jnp.transpose(r2, (1, 0))[:1, :]
            s2 = jax.lax.dot_general(qn2[...], k2, (((1,), (1,)), ((), ())),
                                     preferred_element_type=jnp.float32)
            eb2 = jnp.exp(jnp.clip(s2 * r2row, 0.0, 1.0))
            e2[:, pl.ds(i * BK, BK)] = eb2
            z2[...] += jnp.sum(eb2, axis=1, keepdims=True)

        @pl.when(p == 1)
        def _phase1():
            @pl.when(i == 0)
            def _select():
                a = jax.nn.sigmoid(temp_ref[0, 0])
                c1 = a / z1[...]
                c2 = (1.0 - a) / z2[...]
                attv = e1[...] * c1 + e2[...] * c2
                bits = jax.lax.bitcast_convert_type(attv, jnp.int32)
                mx = jnp.max(attv, axis=1, keepdims=True)
                mn = jnp.min(attv, axis=1, keepdims=True)
                hi0 = jax.lax.bitcast_convert_type(mx, jnp.int32) + 1
                lo0 = jax.lax.bitcast_convert_type(mn, jnp.int32)
                H = N // 2
                base = lo0

                def pack_pairs(x):
                    return x[:, :H] | jax.lax.shift_left(x[:, H:], 16)

                def swar_count(qp, mid):
                    adder = (1 << 15) - mid
                    adder = adder | jax.lax.shift_left(adder, 16)
                    t = qp + adder
                    m = jax.lax.shift_right_logical(t, 15) & 0x00010001
                    cp = jnp.sum(m, axis=1, keepdims=True)
                    return (cp & 0xFFFF) + jax.lax.shift_right_logical(
                        cp, 16)

                def cond(lohi):
                    lo, hi = lohi
                    return jnp.max(hi - lo) > 1

                qa = pack_pairs(jax.lax.shift_right_logical(bits - base, 10))
                qhi0 = jax.lax.shift_right_logical(hi0 - 1 - base, 10) + 1
                qlo0 = jnp.zeros_like(qhi0)

                def stepa(lohi):
                    lo, hi = lohi
                    mid = lo + jax.lax.shift_right_logical(hi - lo, 1)
                    ge = swar_count(qa, mid) >= int(KSEL)
                    return jnp.where(ge, mid, lo), jnp.where(ge, hi, mid)

                qlo, _ = jax.lax.while_loop(cond, stepa, (qlo0, qhi0))
                lob = base + jax.lax.shift_left(qlo, 10)
                hib = jnp.minimum(
                    base + jax.lax.shift_left(qlo + 1, 10), hi0)

                qb = pack_pairs(jnp.clip(bits - lob, 0, 4095))

                def stepb(lohi):
                    lo, hi = lohi
                    mid = lo + jax.lax.shift_right_logical(hi - lo, 1)
                    ge = swar_count(qb, mid - lob) >= int(KSEL)
                    return jnp.where(ge, mid, lo), jnp.where(ge, hi, mid)

                lo, _ = jax.lax.while_loop(cond, stepb, (lob, hib))
                keep = (bits >= lo).astype(jnp.float32)
                colmask = jnp.max(keep, axis=0, keepdims=True)
                att[...] = jax.lax.bitcast_convert_type(
                    bits, jnp.float32) * colmask
                acc[...] = jnp.zeros_like(acc)

            ablk = att[:, pl.ds(i * BK, BK)]
            acc[...] += jax.lax.dot_general(
                ablk, nans_ref[...], (((1,), (0,)), ((), ())),
                preferred_element_type=jnp.float32)

            @pl.when(i == NB - 1)
            def _final():
                oa = acc[...]
                s0 = jnp.sum(oa * o_ref[:, 0, :], axis=1, keepdims=True)
                s1_ = jnp.sum(oa * o_ref[:, 1, :], axis=1, keepdims=True)
                s2_ = jnp.sum(oa * o_ref[:, 2, :], axis=1, keepdims=True)
                out_ref[...] = jnp.concatenate([s0, s1_, s2_], axis=1)

    out = pl.pallas_call(
        body,
        grid=(2, NB),
        in_specs=[
            pl.BlockSpec((B, VD), lambda p, i: (0, 0)),
            pl.BlockSpec((BK, VD), lambda p, i: ((1 - p) * i, 0)),
            pl.BlockSpec((B, LD), lambda p, i: (0, 0)),
            pl.BlockSpec((BK, LD), lambda p, i: ((1 - p) * i, 0)),
            pl.BlockSpec((B, 3, OD), lambda p, i: (0, 0, 0)),
            pl.BlockSpec((BK, OD), lambda p, i: (p * i, 0)),
            pl.BlockSpec((1, 1), lambda p, i: (0, 0)),
        ],
        out_specs=pl.BlockSpec((B, 3), lambda p, i: (0, 0)),
        out_shape=jax.ShapeDtypeStruct((B, 3), jnp.float32),
        scratch_shapes=[
            pltpu.VMEM((B, VD), jnp.float32),
            pltpu.VMEM((B, LD), jnp.float32),
            pltpu.VMEM((B, N), jnp.float32),
            pltpu.VMEM((B, N), jnp.float32),
            pltpu.VMEM((B, 1), jnp.float32),
            pltpu.VMEM((B, 1), jnp.float32),
            pltpu.VMEM((B, N), jnp.float32),
            pltpu.VMEM((B, OD), jnp.float32),
        ],
    )(v, n_feats, aud, n_auds, o, n_answ, temp.reshape(1, 1))
    return out
